# Initial kernel scaffold; baseline (speedup 1.0000x reference)
import jax
import jax.numpy as jnp
from jax.experimental import pallas as pl


def kernel(x, centers, widths, consequents, rules):
    raise NotImplementedError("write your pallas kernel here")


# speedup vs baseline: 7413.5819x; 7413.5819x over previous
import functools

import jax
import jax.numpy as jnp
from jax import lax
from jax.experimental import pallas as pl
from jax.experimental.pallas import tpu as pltpu


def _cumsum_lanes(x):
    n = x.shape[1]
    s = 1
    while s < n:
        shifted = jnp.concatenate(
            [jnp.zeros((x.shape[0], s), x.dtype), x[:, : n - s]], axis=1
        )
        x = x + shifted
        s *= 2
    return x


def _anfis_body(K, M, C, xr_ref, xe_ref, cf_ref, wf_ref, rt5_ref, cons_ref,
                out_ref, nrm_ref, msk_ref):
    BB = xr_ref.shape[0]
    DM = xr_ref.shape[1]
    R = rt5_ref.shape[1]

    xr = xr_ref[...]
    cf = cf_ref[0:1, :]
    wf = wf_ref[0:1, :]
    g = -((xr - cf) ** 2) / (2.0 * wf * wf) + 1e-9

    mm = lax.broadcasted_iota(jnp.int32, (DM, R), 0) % M
    ohf = (rt5_ref[...] == mm).astype(jnp.float32)

    y = jnp.floor(jnp.maximum(g, -120.0) * 262144.0 + 0.5)
    hi = jnp.floor(y * (1.0 / 8192.0))
    lo = y - hi * 8192.0
    s_hi = lax.dot_general(hi, ohf, (((1,), (0,)), ((), ())),
                           preferred_element_type=jnp.float32)
    s_lo = lax.dot_general(lo, ohf, (((1,), (0,)), ((), ())),
                           preferred_element_type=jnp.float32)
    s = s_hi * (1.0 / 32.0) + s_lo * (1.0 / 262144.0)
    fs = jnp.exp(s)

    bits = lax.bitcast_convert_type(fs, jnp.int32)

    def bs_body(_, carry):
        lo, hi = carry
        mid = lo + ((hi - lo + 1) >> 1)
        cnt = jnp.sum((bits >= mid).astype(jnp.int32), axis=1, keepdims=True)
        ge = cnt >= K
        return jnp.where(ge, mid, lo), jnp.where(ge, hi, mid - 1)

    lo0 = jnp.zeros((BB, 1), jnp.int32)
    hi0 = jnp.full((BB, 1), 0x7F800000, jnp.int32)
    t, _ = lax.fori_loop(0, 31, bs_body, (lo0, hi0))

    gt = bits > t
    eq = bits == t
    cnt_gt = jnp.sum(gt.astype(jnp.int32), axis=1, keepdims=True)
    need = K - cnt_gt
    eqi = eq.astype(jnp.int32)
    rank = _cumsum_lanes(eqi) - eqi
    sel = gt | (eq & (rank < need))
    maskf = sel.astype(jnp.float32)

    firing = fs * maskf
    denom = jnp.sum(firing, axis=1, keepdims=True) + 1e-9
    nrm = firing / denom

    msk_ref[...] = maskf
    nrm_ref[...] = nrm

    cons2 = cons_ref[...]
    nj = cons2.shape[1] // C
    csum = cons2[:, 0:C]
    for j in range(1, nj):
        csum = csum + cons2[:, j * C:(j + 1) * C]
    w = lax.dot_general(nrm, csum, (((1,), (0,)), ((), ())),
                        preferred_element_type=jnp.float32)
    sx = jnp.sum(xe_ref[...], axis=1, keepdims=True)
    out_ref[...] = sx * w


def kernel(x, centers, widths, consequents, rules):
    B, D = x.shape
    M = centers.shape[1]
    R = rules.shape[0]
    C = consequents.shape[2]
    DM = D * M
    K = max(1, int(0.2 * R))
    BB = 256

    xr = jnp.repeat(x, M, axis=1)
    xe = jnp.concatenate([x, jnp.ones((B, 1), x.dtype)], axis=1)
    cf = jnp.broadcast_to(centers.reshape(1, DM), (8, DM))
    wf = jnp.broadcast_to(widths.reshape(1, DM), (8, DM))
    rt5 = jnp.repeat(rules.T, M, axis=0)
    cons = consequents.reshape(R, (D + 1) * C)

    grid = (B // BB,)
    out_shape = (
        jax.ShapeDtypeStruct((B, C), jnp.float32),
        jax.ShapeDtypeStruct((B, R), jnp.float32),
        jax.ShapeDtypeStruct((B, R), jnp.float32),
    )
    rule_outputs, normalized, mask = pl.pallas_call(
        functools.partial(_anfis_body, K, M, C),
        grid=grid,
        in_specs=[
            pl.BlockSpec((BB, DM), lambda i: (i, 0)),
            pl.BlockSpec((BB, D + 1), lambda i: (i, 0)),
            pl.BlockSpec((8, DM), lambda i: (0, 0)),
            pl.BlockSpec((8, DM), lambda i: (0, 0)),
            pl.BlockSpec((DM, R), lambda i: (0, 0)),
            pl.BlockSpec((R, (D + 1) * C), lambda i: (0, 0)),
        ],
        out_specs=(
            pl.BlockSpec((BB, C), lambda i: (i, 0)),
            pl.BlockSpec((BB, R), lambda i: (i, 0)),
            pl.BlockSpec((BB, R), lambda i: (i, 0)),
        ),
        out_shape=out_shape,
    )(xr, xe, cf, wf, rt5, cons)
    return (rule_outputs, normalized, mask)


# speedup vs baseline: 8605.0818x; 1.1607x over previous
import functools

import jax
import jax.numpy as jnp
from jax import lax
from jax.experimental import pallas as pl


def _fs_topk_body(M, K, xr_ref, cf_ref, wf_ref, rt5_ref, nrm_ref, msk_ref):
    DM = xr_ref.shape[1]
    R = rt5_ref.shape[1]
    BB = xr_ref.shape[0]
    xr = xr_ref[...]
    cf = cf_ref[0:1, :]
    wf = wf_ref[0:1, :]
    g = -((xr - cf) ** 2) / (2.0 * wf * wf) + 1e-9

    mm = lax.broadcasted_iota(jnp.int32, (DM, R), 0) % M
    ohf = (rt5_ref[...] == mm).astype(jnp.float32)

    y = jnp.floor(jnp.maximum(g, -120.0) * 262144.0 + 0.5)
    hi_h = jnp.floor(y * (1.0 / 8192.0))
    lo_h = y - hi_h * 8192.0
    s_hi = lax.dot_general(hi_h, ohf, (((1,), (0,)), ((), ())),
                           preferred_element_type=jnp.float32)
    s_lo = lax.dot_general(lo_h, ohf, (((1,), (0,)), ((), ())),
                           preferred_element_type=jnp.float32)
    s = s_hi * (1.0 / 32.0) + s_lo * (1.0 / 262144.0)
    fs = jnp.exp(s)

    bits = lax.bitcast_convert_type(fs, jnp.int32)
    lo0 = jnp.min(bits, axis=1, keepdims=True)
    hi0 = jnp.max(bits, axis=1, keepdims=True)

    def bs_body(_, c):
        lo, hi = c
        lv = lo < hi
        mid = lo + ((hi - lo + 1) >> 1)
        cnt = jnp.sum((bits >= mid).astype(jnp.int32), axis=1, keepdims=True)
        ge = cnt >= K
        new_lo = jnp.where(lv & ge, mid, lo)
        new_hi = jnp.where(lv & (~ge), mid - 1, hi)
        return new_lo, new_hi

    t, _ = lax.fori_loop(0, 31, bs_body, (lo0, hi0))

    gt = bits > t
    eq = bits == t
    eqi = eq.astype(jnp.int32)
    navail = K - jnp.sum(gt.astype(jnp.int32), axis=1, keepdims=True)

    cs = eqi
    sh = 1
    while sh < R:
        cs = cs + jnp.concatenate(
            [jnp.zeros((BB, sh), jnp.int32), cs[:, :R - sh]], axis=1)
        sh *= 2
    excl = cs - eqi

    mask = (gt | (eq & (excl < navail))).astype(jnp.float32)
    firing = fs * mask
    denom = jnp.sum(firing, axis=1, keepdims=True) + 1e-9
    nrm_ref[...] = firing / denom
    msk_ref[...] = mask


def _fs_topk(x, centers, widths, rules, K):
    B, D = x.shape
    M = centers.shape[1]
    R = rules.shape[0]
    DM = D * M
    BB = 256
    xr = jnp.repeat(x, M, axis=1)
    cf = jnp.broadcast_to(centers.reshape(1, DM), (8, DM))
    wf = jnp.broadcast_to(widths.reshape(1, DM), (8, DM))
    rt5 = jnp.repeat(rules.T, M, axis=0)
    return pl.pallas_call(
        functools.partial(_fs_topk_body, M, K),
        grid=(B // BB,),
        in_specs=[
            pl.BlockSpec((BB, DM), lambda i: (i, 0)),
            pl.BlockSpec((8, DM), lambda i: (0, 0)),
            pl.BlockSpec((8, DM), lambda i: (0, 0)),
            pl.BlockSpec((DM, R), lambda i: (0, 0)),
        ],
        out_specs=[
            pl.BlockSpec((BB, R), lambda i: (i, 0)),
            pl.BlockSpec((BB, R), lambda i: (i, 0)),
        ],
        out_shape=[
            jax.ShapeDtypeStruct((B, R), jnp.float32),
            jax.ShapeDtypeStruct((B, R), jnp.float32),
        ],
    )(xr, cf, wf, rt5)


def _combine_body(C, nrm_ref, xe_ref, cons_ref, out_ref):
    cons2 = cons_ref[...]
    nj = cons2.shape[1] // C
    csum = cons2[:, 0:C]
    for j in range(1, nj):
        csum = csum + cons2[:, j * C:(j + 1) * C]
    w = lax.dot_general(nrm_ref[...], csum, (((1,), (0,)), ((), ())),
                        preferred_element_type=jnp.float32)
    sx = jnp.sum(xe_ref[...], axis=1, keepdims=True)
    out_ref[...] = sx * w


def _combine(nrm, x, consequents):
    B, R = nrm.shape
    D = x.shape[1]
    C = consequents.shape[2]
    xe = jnp.concatenate([x, jnp.ones((B, 1), x.dtype)], axis=1)
    cons = consequents.reshape(R, (D + 1) * C)
    return pl.pallas_call(
        functools.partial(_combine_body, C),
        in_specs=[
            pl.BlockSpec((B, R), lambda: (0, 0)),
            pl.BlockSpec((B, D + 1), lambda: (0, 0)),
            pl.BlockSpec((R, (D + 1) * C), lambda: (0, 0)),
        ],
        out_specs=pl.BlockSpec((B, C), lambda: (0, 0)),
        out_shape=jax.ShapeDtypeStruct((B, C), jnp.float32),
    )(nrm, xe, cons)


def kernel(x, centers, widths, consequents, rules):
    R = rules.shape[0]
    K = max(1, int(0.2 * R))
    normalized, mask = _fs_topk(x, centers, widths, rules, K)
    rule_outputs = _combine(normalized, x, consequents)
    return (rule_outputs, normalized, mask)
